# Initial kernel scaffold; baseline (speedup 1.0000x reference)
import jax
import jax.numpy as jnp
from jax.experimental import pallas as pl


def kernel(pred, target):
    raise NotImplementedError("write your pallas kernel here")


# speedup vs baseline: 10103.9988x; 10103.9988x over previous
import functools

import jax
import jax.numpy as jnp
from jax.experimental import pallas as pl
from jax.experimental.pallas import tpu as pltpu

_N = 2048
_ROWS = 4096
_BLOCK_ROWS = 256


def _bitonic_sort_rows(x, lane):
    n = x.shape[1]
    log_n = n.bit_length() - 1
    for kl in range(1, log_n + 1):
        k = 1 << kl
        for jl in range(kl - 1, -1, -1):
            j = 1 << jl
            fwd = pltpu.roll(x, n - j, axis=1)
            bwd = pltpu.roll(x, j, axis=1)
            is_lo = (lane & j) == 0
            partner = jnp.where(is_lo, fwd, bwd)
            keep_min = ((lane & k) == 0) == is_lo
            x = jnp.where(keep_min, jnp.minimum(x, partner),
                          jnp.maximum(x, partner))
    return x


def _w1_kernel(pred_ref, target_ref, out_ref):
    lane = jax.lax.broadcasted_iota(jnp.int32, (_BLOCK_ROWS, _N), 1)
    su = _bitonic_sort_rows(pred_ref[...], lane)
    sv = _bitonic_sort_rows(target_ref[...], lane)
    partial = jnp.sum(jnp.abs(su - sv)).reshape(1, 1)

    @pl.when(pl.program_id(0) == 0)
    def _init():
        out_ref[...] = jnp.zeros((1, 1), jnp.float32)

    out_ref[...] += partial


@jax.jit
def kernel(pred, target):
    num_blocks = _ROWS // _BLOCK_ROWS
    total = pl.pallas_call(
        _w1_kernel,
        grid=(num_blocks,),
        in_specs=[
            pl.BlockSpec((_BLOCK_ROWS, _N), lambda i: (i, 0)),
            pl.BlockSpec((_BLOCK_ROWS, _N), lambda i: (i, 0)),
        ],
        out_specs=pl.BlockSpec((1, 1), lambda i: (0, 0)),
        out_shape=jax.ShapeDtypeStruct((1, 1), jnp.float32),
    )(pred, target)
    return total[0, 0] / jnp.float32(_ROWS * _N)
